# baseline (speedup 1.0000x reference)
import functools
import jax
import jax.numpy as jnp
from jax import lax
from jax.experimental import pallas as pl
from jax.experimental.pallas import tpu as pltpu
from jax.experimental.pallas import tpu_sc as plsc

_OUTSIZE = 512


def kernel(x):
    assert x.ndim == 2
    n = x.shape[0]
    d = min(x.shape[1], _OUTSIZE)
    info = plsc.get_sparse_core_info()
    nw = info.num_cores * info.num_subcores
    rows_per_w = n // nw
    br = 32
    mesh = plsc.VectorSubcoreMesh(core_axis_name="c", subcore_axis_name="s")

    @functools.partial(
        pl.kernel,
        mesh=mesh,
        out_type=jax.ShapeDtypeStruct((n, d), jnp.float32),
        scratch_types=[pltpu.VMEM((br, d), jnp.float32)],
    )
    def _sc_zero(out_hbm, zbuf):
        z16 = jnp.zeros((16,), jnp.float32)

        def fill_row(i, carry):
            def fill_lane(j, c):
                zbuf[i, pl.ds(j * 16, 16)] = z16
                return c

            return lax.fori_loop(0, d // 16, fill_lane, carry)

        lax.fori_loop(0, br, fill_row, 0)

        wid = lax.axis_index("s") * info.num_cores + lax.axis_index("c")
        base = wid * rows_per_w

        def copy_blk(b, carry):
            pltpu.sync_copy(zbuf, out_hbm.at[pl.ds(base + b * br, br)])
            return carry

        lax.fori_loop(0, rows_per_w // br, copy_blk, 0)

    return _sc_zero()


# speedup vs baseline: 2.7519x; 2.7519x over previous
import jax
import jax.numpy as jnp
from jax.experimental import pallas as pl

_OUTSIZE = 512


def _zero_fill(o_ref):
    o_ref[...] = jnp.zeros_like(o_ref)


def kernel(x):
    assert x.ndim == 2
    n = x.shape[0]
    d = min(x.shape[1], _OUTSIZE)
    block_rows = min(n, 1024)
    return pl.pallas_call(
        _zero_fill,
        grid=(n // block_rows,),
        out_specs=pl.BlockSpec((block_rows, d), lambda i: (i, 0)),
        out_shape=jax.ShapeDtypeStruct((n, d), jnp.float32),
    )()


# speedup vs baseline: 3.1175x; 1.1329x over previous
import jax
import jax.numpy as jnp
from jax.experimental import pallas as pl

_OUTSIZE = 512


def _zero_fill(o_ref):
    o_ref[...] = jnp.zeros_like(o_ref)


def kernel(x):
    assert x.ndim == 2
    n = x.shape[0]
    d = min(x.shape[1], _OUTSIZE)
    block_rows = min(n, 2048)
    return pl.pallas_call(
        _zero_fill,
        grid=(n // block_rows,),
        out_specs=pl.BlockSpec((block_rows, d), lambda i: (i, 0)),
        out_shape=jax.ShapeDtypeStruct((n, d), jnp.float32),
    )()
